# baseline (speedup 1.0000x reference)
import functools

import jax
import jax.numpy as jnp
from jax import lax
from jax.experimental import pallas as pl
from jax.experimental.pallas import tpu as pltpu
from jax.experimental.pallas import tpu_sc as plsc

N = 44400
E = 710400
G = 400
F0 = 111

NC = 2
NS = 16
NT = NC * NS

NP = 45056
EP = 720896
GP = 512
PAD_NODE = N

NB_E = EP // 128 // NT
MSLICE = NP // NS
GSLICE = GP // NS
NGRP = NP // 1024

RB = 2816
NRB = NP // RB

_f32 = jnp.float32


def _mesh():
  return plsc.VectorSubcoreMesh(core_axis_name="c", subcore_axis_name="s")


def _deg_body(dst_hbm, zeros_hbm, out_hbm, acc, didx):
  ci = lax.axis_index("c")
  si = lax.axis_index("s")
  wid = ci * NS + si
  ept = EP // NT
  pltpu.sync_copy(zeros_hbm, acc)
  pltpu.sync_copy(dst_hbm.at[pl.ds(wid * ept, ept)], didx)
  ones = jnp.ones((16,), _f32)

  def body(j, carry):
    idx = didx[pl.ds(j * 16, 16)]
    plsc.addupdate_scatter(acc, [idx], ones)
    return carry

  lax.fori_loop(0, ept // 16, body, 0)
  pltpu.sync_copy(acc, out_hbm.at[pl.ds(wid * NP, NP)])


def _deg(dst1d, zeros_np):
  k = pl.kernel(
      _deg_body,
      out_type=jax.ShapeDtypeStruct((NT * NP,), _f32),
      mesh=_mesh(),
      compiler_params=pltpu.CompilerParams(needs_layout_passes=False, use_tc_tiling_on_sc=False),
      scratch_types=[
          pltpu.VMEM((NP,), _f32),
          pltpu.VMEM((EP // NT,), jnp.int32),
      ],
  )
  return k(dst1d, zeros_np)


NB0 = 240
NB1 = 112
SLAB = 16


def _make_agg(fc, nchunks):
  def body(*refs):
    us = refs[:nchunks]
    src_hbm, dst_hbm = refs[nchunks:nchunks + 2]
    outs = refs[nchunks + 2:2 * nchunks + 2]
    (sixa, dixa, sixb, dixb, rows_a, rows_b, zbuf, acc,
     isem_a, isem_b, gsem_a, gsem_b, ssem) = refs[2 * nchunks + 2:]
    ci = lax.axis_index("c")
    si = lax.axis_index("s")
    nb = jnp.where(ci == 0, NB0, NB1)
    row0 = ci * NS * NB0 + si * nb
    nslab = nb // SLAB
    nq = jnp.where(ci == 0, (NB0 // SLAB + 1) // 2, (NB1 // SLAB + 1) // 2)

    for i in range(128):
      zbuf[i, :] = jnp.zeros((16,), _f32)

    def ifire(p, six, dix, isem):
      pltpu.async_copy(src_hbm.at[pl.ds(row0 + p * SLAB, SLAB), :], six, isem)
      pltpu.async_copy(dst_hbm.at[pl.ds(row0 + p * SLAB, SLAB), :], dix, isem)

    def iwait(p, six, dix, isem):
      pltpu.make_async_copy(src_hbm.at[pl.ds(row0 + p * SLAB, SLAB), :],
                            six, isem).wait()
      pltpu.make_async_copy(dst_hbm.at[pl.ds(row0 + p * SLAB, SLAB), :],
                            dix, isem).wait()

    for c in range(nchunks):
      u = us[c]

      def slab_run(p, six, dix, isem, osix, odix, oisem):
        iwait(p, six, dix, isem)

        @pl.when(p + 1 < nslab)
        def _():
          ifire(p + 1, osix, odix, oisem)

        for j in range(8):
          pltpu.async_copy(u.at[six.at[j]],
                           rows_a.at[pl.ds(j * 128, 128), :], gsem_a)
        for j in range(8):
          pltpu.async_copy(u.at[six.at[8 + j]],
                           rows_b.at[pl.ds(j * 128, 128), :], gsem_b)
        for j in range(8):
          pltpu.make_async_copy(u.at[six.at[j]],
                                rows_a.at[pl.ds(j * 128, 128), :],
                                gsem_a).wait()
        for j in range(8):
          pltpu.async_copy(rows_a.at[pl.ds(j * 128, 128), :],
                           acc.at[dix.at[j]], ssem, add=True)
        for j in range(8):
          pltpu.make_async_copy(u.at[six.at[8 + j]],
                                rows_b.at[pl.ds(j * 128, 128), :],
                                gsem_b).wait()
        for j in range(8):
          pltpu.async_copy(rows_b.at[pl.ds(j * 128, 128), :],
                           acc.at[dix.at[8 + j]], ssem, add=True)
        for j in range(8):
          pltpu.make_async_copy(rows_a.at[pl.ds(j * 128, 128), :],
                                acc.at[dix.at[j]], ssem).wait()
        for j in range(8):
          pltpu.make_async_copy(rows_b.at[pl.ds(j * 128, 128), :],
                                acc.at[dix.at[8 + j]], ssem).wait()

      def zrun(i, carry):
        pltpu.sync_copy(zbuf, acc.at[pl.ds(si * MSLICE + i * 128, 128), :])
        return carry

      lax.fori_loop(0, MSLICE // 128, zrun, 0)
      plsc.subcore_barrier()
      ifire(0, sixa, dixa, isem_a)

      def qpair(q, carry):
        slab_run(2 * q, sixa, dixa, isem_a, sixb, dixb, isem_b)

        @pl.when(2 * q + 1 < nslab)
        def _():
          slab_run(2 * q + 1, sixb, dixb, isem_b, sixa, dixa, isem_a)
        return carry

      lax.fori_loop(0, nq, qpair, 0)
      plsc.subcore_barrier()
      pltpu.sync_copy(acc.at[pl.ds(si * MSLICE, MSLICE), :],
                      outs[c].at[ci, pl.ds(si * MSLICE, MSLICE), :])
      if c + 1 < nchunks:
        plsc.subcore_barrier()

  def run(us, src2d, dst2d):
    k = pl.kernel(
        body,
        out_type=[jax.ShapeDtypeStruct((NC, NP, fc), _f32)
                  for _ in range(nchunks)],
        mesh=_mesh(),
        compiler_params=pltpu.CompilerParams(needs_layout_passes=False, use_tc_tiling_on_sc=False),
        scratch_types=[
            pltpu.VMEM((SLAB, 128), jnp.int32),
            pltpu.VMEM((SLAB, 128), jnp.int32),
            pltpu.VMEM((SLAB, 128), jnp.int32),
            pltpu.VMEM((SLAB, 128), jnp.int32),
            pltpu.VMEM((8 * 128, fc), _f32),
            pltpu.VMEM((8 * 128, fc), _f32),
            pltpu.VMEM((128, fc), _f32),
            pltpu.VMEM_SHARED((NP, fc), _f32),
            pltpu.SemaphoreType.DMA,
            pltpu.SemaphoreType.DMA,
            pltpu.SemaphoreType.DMA,
            pltpu.SemaphoreType.DMA,
            pltpu.SemaphoreType.DMA,
        ],
    )
    return k(*us, src2d, dst2d)

  return run


def _pool_body(xp_h, o1_h, o2_h, o3_h, b_hbm, zx_h, z1_h, z2_h, z3_h,
               px, p1, p2, p3,
               bidx, rx, r1, r2, r3, accx, acc1, acc2, acc3, sem):
  ci = lax.axis_index("c")
  si = lax.axis_index("s")
  wid = ci * NS + si
  g0 = si * GSLICE
  pltpu.sync_copy(zx_h, accx.at[pl.ds(g0, GSLICE), :])
  pltpu.sync_copy(z1_h, acc1.at[pl.ds(g0, GSLICE), :])
  pltpu.sync_copy(z2_h, acc2.at[pl.ds(g0, GSLICE), :])
  pltpu.sync_copy(z3_h, acc3.at[pl.ds(g0, GSLICE), :])
  plsc.subcore_barrier()

  for gi in range(2):
    grp = wid + gi * NT

    @pl.when(grp < NGRP)
    def _():
      pltpu.sync_copy(b_hbm.at[pl.ds(grp * 8, 8), :], bidx)

      def body(j, carry):
        n0 = grp * 1024 + j * 128
        pltpu.sync_copy(xp_h.at[pl.ds(n0, 128), :], rx)
        pltpu.sync_copy(o1_h.at[pl.ds(n0, 128), :], r1)
        pltpu.sync_copy(o2_h.at[pl.ds(n0, 128), :], r2)
        pltpu.sync_copy(o3_h.at[pl.ds(n0, 128), :], r3)
        idx = bidx.at[j]
        pltpu.sync_copy(rx, accx.at[idx], add=True)
        pltpu.sync_copy(r1, acc1.at[idx], add=True)
        pltpu.sync_copy(r2, acc2.at[idx], add=True)
        pltpu.sync_copy(r3, acc3.at[idx], add=True)
        return carry

      lax.fori_loop(0, 8, body, 0)

  plsc.subcore_barrier()
  pltpu.sync_copy(accx.at[pl.ds(g0, GSLICE), :],
                  px.at[ci, pl.ds(g0, GSLICE), :])
  pltpu.sync_copy(acc1.at[pl.ds(g0, GSLICE), :],
                  p1.at[ci, pl.ds(g0, GSLICE), :])
  pltpu.sync_copy(acc2.at[pl.ds(g0, GSLICE), :],
                  p2.at[ci, pl.ds(g0, GSLICE), :])
  pltpu.sync_copy(acc3.at[pl.ds(g0, GSLICE), :],
                  p3.at[ci, pl.ds(g0, GSLICE), :])


def _pool(xp, o1, o2, o3, b2d, zx, z1, z2, z3):
  k = pl.kernel(
      _pool_body,
      out_type=[
          jax.ShapeDtypeStruct((NC, GP, 112), _f32),
          jax.ShapeDtypeStruct((NC, GP, 64), _f32),
          jax.ShapeDtypeStruct((NC, GP, 32), _f32),
          jax.ShapeDtypeStruct((NC, GP, 16), _f32),
      ],
      mesh=_mesh(),
      compiler_params=pltpu.CompilerParams(needs_layout_passes=False, use_tc_tiling_on_sc=False),
      scratch_types=[
          pltpu.VMEM((8, 128), jnp.int32),
          pltpu.VMEM((128, 112), _f32),
          pltpu.VMEM((128, 64), _f32),
          pltpu.VMEM((128, 32), _f32),
          pltpu.VMEM((128, 16), _f32),
          pltpu.VMEM_SHARED((GP, 112), _f32),
          pltpu.VMEM_SHARED((GP, 64), _f32),
          pltpu.VMEM_SHARED((GP, 32), _f32),
          pltpu.VMEM_SHARED((GP, 16), _f32),
          pltpu.SemaphoreType.DMA,
      ],
  )
  return k(xp, o1, o2, o3, b2d, zx, z1, z2, z3)


def _m1_body(x_ref, w_ref, o_ref):
  o_ref[...] = jnp.dot(x_ref[...], w_ref[...], preferred_element_type=_f32)


def _m1(xp, w1tp):
  return pl.pallas_call(
      _m1_body,
      grid=(NRB,),
      in_specs=[
          pl.BlockSpec((RB, 112), lambda i: (i, 0)),
          pl.BlockSpec((112, 64), lambda i: (0, 0)),
      ],
      out_specs=pl.BlockSpec((RB, 64), lambda i: (i, 0)),
      out_shape=jax.ShapeDtypeStruct((NP, 64), _f32),
  )(xp, w1tp)


def _dsum_body(dp_ref, o_ref):
  o_ref[...] = jnp.sum(dp_ref[...], axis=0)


def _dsum(degp):
  lb = 4096
  return pl.pallas_call(
      _dsum_body,
      grid=(NP // 4096,),
      in_specs=[pl.BlockSpec((NT, lb), lambda i: (0, i))],
      out_specs=pl.BlockSpec((lb,), lambda i: (i,)),
      out_shape=jax.ShapeDtypeStruct((NP,), _f32),
  )(degp)


def _u1prep_body(dcol_ref, h1_ref, dinv_ref, dinv2_ref,
                 ua_ref, ub_ref, uc_ref, ud_ref):
  d = dcol_ref[...] + 1.0
  di = lax.rsqrt(d)
  dinv_ref[...] = di
  dinv2_ref[...] = 1.0 / d
  u = h1_ref[...] * di
  ua_ref[...] = u[:, 0:16]
  ub_ref[...] = u[:, 16:32]
  uc_ref[...] = u[:, 32:48]
  ud_ref[...] = u[:, 48:64]


def _u1prep(dcol, h1):
  return pl.pallas_call(
      _u1prep_body,
      grid=(NRB,),
      in_specs=[
          pl.BlockSpec((RB, 1), lambda i: (i, 0)),
          pl.BlockSpec((RB, 64), lambda i: (i, 0)),
      ],
      out_specs=[
          pl.BlockSpec((RB, 1), lambda i: (i, 0)),
          pl.BlockSpec((RB, 1), lambda i: (i, 0)),
          pl.BlockSpec((RB, 16), lambda i: (i, 0)),
          pl.BlockSpec((RB, 16), lambda i: (i, 0)),
          pl.BlockSpec((RB, 16), lambda i: (i, 0)),
          pl.BlockSpec((RB, 16), lambda i: (i, 0)),
      ],
      out_shape=[
          jax.ShapeDtypeStruct((NP, 1), _f32),
          jax.ShapeDtypeStruct((NP, 1), _f32),
          jax.ShapeDtypeStruct((NP, 16), _f32),
          jax.ShapeDtypeStruct((NP, 16), _f32),
          jax.ShapeDtypeStruct((NP, 16), _f32),
          jax.ShapeDtypeStruct((NP, 16), _f32),
      ],
  )(dcol, h1)


def _epi1_body(a0_ref, a1_ref, a2_ref, a3_ref, h1_ref, dinv_ref, dinv2_ref,
               b_ref, w_ref, o1_ref, h2_ref, u2a_ref, u2b_ref):
  agg = jnp.concatenate(
      [jnp.sum(a0_ref[...], axis=0), jnp.sum(a1_ref[...], axis=0),
       jnp.sum(a2_ref[...], axis=0), jnp.sum(a3_ref[...], axis=0)], axis=1)
  o1 = jnp.maximum(
      dinv_ref[...] * agg + dinv2_ref[...] * h1_ref[...] + b_ref[...], 0.0)
  o1_ref[...] = o1
  h2 = jnp.dot(o1, w_ref[...], preferred_element_type=_f32)
  h2_ref[...] = h2
  u2 = h2 * dinv_ref[...]
  u2a_ref[...] = u2[:, 0:16]
  u2b_ref[...] = u2[:, 16:32]


def _epi1(aggs, h1, dinv, dinv2, b1r, w2t):
  return pl.pallas_call(
      _epi1_body,
      grid=(NRB,),
      in_specs=[
          pl.BlockSpec((NC, RB, 16), lambda i: (0, i, 0)),
          pl.BlockSpec((NC, RB, 16), lambda i: (0, i, 0)),
          pl.BlockSpec((NC, RB, 16), lambda i: (0, i, 0)),
          pl.BlockSpec((NC, RB, 16), lambda i: (0, i, 0)),
          pl.BlockSpec((RB, 64), lambda i: (i, 0)),
          pl.BlockSpec((RB, 1), lambda i: (i, 0)),
          pl.BlockSpec((RB, 1), lambda i: (i, 0)),
          pl.BlockSpec((1, 64), lambda i: (0, 0)),
          pl.BlockSpec((64, 32), lambda i: (0, 0)),
      ],
      out_specs=[
          pl.BlockSpec((RB, 64), lambda i: (i, 0)),
          pl.BlockSpec((RB, 32), lambda i: (i, 0)),
          pl.BlockSpec((RB, 16), lambda i: (i, 0)),
          pl.BlockSpec((RB, 16), lambda i: (i, 0)),
      ],
      out_shape=[
          jax.ShapeDtypeStruct((NP, 64), _f32),
          jax.ShapeDtypeStruct((NP, 32), _f32),
          jax.ShapeDtypeStruct((NP, 16), _f32),
          jax.ShapeDtypeStruct((NP, 16), _f32),
      ],
  )(*aggs, h1, dinv, dinv2, b1r, w2t)


def _epi2_body(a0_ref, a1_ref, h2_ref, dinv_ref, dinv2_ref, b_ref, w_ref,
               o2_ref, h3_ref, u3_ref):
  agg = jnp.concatenate(
      [jnp.sum(a0_ref[...], axis=0), jnp.sum(a1_ref[...], axis=0)], axis=1)
  o2 = jnp.maximum(
      dinv_ref[...] * agg + dinv2_ref[...] * h2_ref[...] + b_ref[...], 0.0)
  o2_ref[...] = o2
  h3 = jnp.dot(o2, w_ref[...], preferred_element_type=_f32)
  h3_ref[...] = h3
  u3_ref[...] = h3 * dinv_ref[...]


def _epi2(aggs, h2, dinv, dinv2, b2r, w3t):
  return pl.pallas_call(
      _epi2_body,
      grid=(NRB,),
      in_specs=[
          pl.BlockSpec((NC, RB, 16), lambda i: (0, i, 0)),
          pl.BlockSpec((NC, RB, 16), lambda i: (0, i, 0)),
          pl.BlockSpec((RB, 32), lambda i: (i, 0)),
          pl.BlockSpec((RB, 1), lambda i: (i, 0)),
          pl.BlockSpec((RB, 1), lambda i: (i, 0)),
          pl.BlockSpec((1, 32), lambda i: (0, 0)),
          pl.BlockSpec((32, 16), lambda i: (0, 0)),
      ],
      out_specs=[
          pl.BlockSpec((RB, 32), lambda i: (i, 0)),
          pl.BlockSpec((RB, 16), lambda i: (i, 0)),
          pl.BlockSpec((RB, 16), lambda i: (i, 0)),
      ],
      out_shape=[
          jax.ShapeDtypeStruct((NP, 32), _f32),
          jax.ShapeDtypeStruct((NP, 16), _f32),
          jax.ShapeDtypeStruct((NP, 16), _f32),
      ],
  )(*aggs, h2, dinv, dinv2, b2r, w3t)


def _epi3_body(a_ref, h3_ref, dinv_ref, dinv2_ref, b_ref, o3_ref):
  agg = jnp.sum(a_ref[...], axis=0)
  o3_ref[...] = jnp.maximum(
      dinv_ref[...] * agg + dinv2_ref[...] * h3_ref[...] + b_ref[...], 0.0)


def _epi3(a, h3, dinv, dinv2, b3r):
  return pl.pallas_call(
      _epi3_body,
      grid=(NRB,),
      in_specs=[
          pl.BlockSpec((NC, RB, 16), lambda i: (0, i, 0)),
          pl.BlockSpec((RB, 16), lambda i: (i, 0)),
          pl.BlockSpec((RB, 1), lambda i: (i, 0)),
          pl.BlockSpec((RB, 1), lambda i: (i, 0)),
          pl.BlockSpec((1, 16), lambda i: (0, 0)),
      ],
      out_specs=pl.BlockSpec((RB, 16), lambda i: (i, 0)),
      out_shape=jax.ShapeDtypeStruct((NP, 16), _f32),
  )(a, h3, dinv, dinv2, b3r)


def _head_body(px_ref, p1_ref, p2_ref, p3_ref, bng_ref, bnb_ref, bnm_ref,
               bnv_ref, f1w_ref, f1b_ref, f2w_ref, f2b_ref, out_ref):
  px = px_ref[...]
  sx = px[0] + px[1]
  s1 = p1_ref[...][0] + p1_ref[...][1]
  s2 = p2_ref[...][0] + p2_ref[...][1]
  s3 = p3_ref[...][0] + p3_ref[...][1]
  cnt = jnp.maximum(sx[:, 111:112], 1.0)
  agg = jnp.concatenate([sx[:, :111], s1, s2, s3], axis=1) / cnt
  a = ((agg - bnm_ref[...]) * lax.rsqrt(bnv_ref[...] + 1e-5) * bng_ref[...]
       + bnb_ref[...])
  z = jnp.maximum(
      jnp.dot(a, f1w_ref[...], preferred_element_type=_f32) + f1b_ref[...],
      0.0)
  z = jnp.dot(z, f2w_ref[...], preferred_element_type=_f32) + f2b_ref[...]
  out_ref[...] = jax.nn.sigmoid(z[:G, :])


def _head(px, p1, p2, p3, bng, bnb, bnm, bnv, f1w, f1b, f2w, f2b):
  return pl.pallas_call(
      _head_body,
      out_shape=jax.ShapeDtypeStruct((G, 1), _f32),
  )(px, p1, p2, p3, bng, bnb, bnm, bnv, f1w, f1b, f2w, f2b)


def kernel(x, edge_index, batches, W1, b1, W2, b2, W3, b3,
           bn_gamma, bn_beta, bn_mean, bn_var, fc1_W, fc1_b, fc2_W, fc2_b):
  xp = jnp.pad(
      jnp.concatenate([x, jnp.ones((N, 1), _f32)], axis=1),
      ((0, NP - N), (0, 0)))
  src1 = jnp.pad(edge_index[0], (0, EP - E), constant_values=PAD_NODE)
  dst1 = jnp.pad(edge_index[1], (0, EP - E), constant_values=PAD_NODE)
  src2d = src1.reshape(EP // 128, 128)
  dst2d = dst1.reshape(EP // 128, 128)
  b2d = jnp.pad(batches, (0, NP - N), constant_values=G).reshape(NP // 128, 128)

  w1tp = jnp.pad(W1.T, ((0, 1), (0, 0)))
  w2t = W2.T
  w3t = W3.T
  b1r = b1.reshape(1, 64)
  b2r = b2.reshape(1, 32)
  b3r = b3.reshape(1, 16)
  f1w = fc1_W.T
  f1b = fc1_b.reshape(1, 64)
  f2w = fc2_W.T
  f2b = fc2_b.reshape(1, 1)
  bng = bn_gamma.reshape(1, 223)
  bnb = bn_beta.reshape(1, 223)
  bnm = bn_mean.reshape(1, 223)
  bnv = bn_var.reshape(1, 223)

  zeros_np = jnp.zeros((NP,), _f32)
  zgx = jnp.zeros((GSLICE, 112), _f32)
  zg1 = jnp.zeros((GSLICE, 64), _f32)
  zg2 = jnp.zeros((GSLICE, 32), _f32)
  zg3 = jnp.zeros((GSLICE, 16), _f32)

  h1 = _m1(xp, w1tp)
  degp = _deg(dst1, zeros_np)
  dsum = _dsum(degp.reshape(NT, NP))
  dinv, dinv2, u1a, u1b, u1c, u1d = _u1prep(dsum.reshape(NP, 1), h1)

  agg1 = _make_agg(16, 4)([u1a, u1b, u1c, u1d], src2d, dst2d)
  o1, h2, u2a, u2b = _epi1(agg1, h1, dinv, dinv2, b1r, w2t)

  agg2 = _make_agg(16, 2)([u2a, u2b], src2d, dst2d)
  o2, h3, u3 = _epi2(agg2, h2, dinv, dinv2, b2r, w3t)

  agg3 = _make_agg(16, 1)([u3], src2d, dst2d)
  o3 = _epi3(agg3[0], h3, dinv, dinv2, b3r)

  px, p1, p2, p3 = _pool(xp, o1, o2, o3, b2d, zgx, zg1, zg2, zg3)
  return _head(px, p1, p2, p3, bng, bnb, bnm, bnv, f1w, f1b, f2w, f2b)


# speedup vs baseline: 1.0672x; 1.0672x over previous
import functools

import jax
import jax.numpy as jnp
from jax import lax
from jax.experimental import pallas as pl
from jax.experimental.pallas import tpu as pltpu
from jax.experimental.pallas import tpu_sc as plsc

N = 44400
E = 710400
G = 400
F0 = 111

NC = 2
NS = 16
NT = NC * NS

NP = 45056
EP = 720896
GP = 512
PAD_NODE = N

NB_E = EP // 128 // NT
MSLICE = NP // NS
GSLICE = GP // NS
NGRP = NP // 1024

RB = 2816
NRB = NP // RB

_f32 = jnp.float32


def _mesh():
  return plsc.VectorSubcoreMesh(core_axis_name="c", subcore_axis_name="s")


def _deg_body(dst_hbm, zeros_hbm, out_hbm, acc, didx):
  ci = lax.axis_index("c")
  si = lax.axis_index("s")
  wid = ci * NS + si
  ept = EP // NT
  pltpu.sync_copy(zeros_hbm, acc)
  pltpu.sync_copy(dst_hbm.at[pl.ds(wid * ept, ept)], didx)
  ones = jnp.ones((16,), _f32)

  def body(j, carry):
    idx = didx[pl.ds(j * 16, 16)]
    plsc.addupdate_scatter(acc, [idx], ones)
    return carry

  lax.fori_loop(0, ept // 16, body, 0)
  pltpu.sync_copy(acc, out_hbm.at[pl.ds(wid * NP, NP)])


def _deg(dst1d, zeros_np):
  k = pl.kernel(
      _deg_body,
      out_type=jax.ShapeDtypeStruct((NT * NP,), _f32),
      mesh=_mesh(),
      compiler_params=pltpu.CompilerParams(needs_layout_passes=False, use_tc_tiling_on_sc=False),
      scratch_types=[
          pltpu.VMEM((NP,), _f32),
          pltpu.VMEM((EP // NT,), jnp.int32),
      ],
  )
  return k(dst1d, zeros_np)


K_BANK = 4
NB0 = 256
NB1 = 96


def _make_agg(fc, nchunks):
  def body(*refs):
    us = refs[:nchunks]
    src_hbm, dst_hbm = refs[nchunks:nchunks + 2]
    outs = refs[nchunks + 2:2 * nchunks + 2]
    (sidx, didx, rows_a, rows_b, zbuf, acc,
     gsem_a, gsem_b, ssem) = refs[2 * nchunks + 2:]
    ci = lax.axis_index("c")
    si = lax.axis_index("s")
    nb = jnp.where(ci == 0, NB0, NB1)
    row0 = ci * NS * NB0 + si * nb
    npairs = jnp.where(ci == 0, NB0 // (2 * K_BANK), NB1 // (2 * K_BANK))
    pltpu.sync_copy(src_hbm.at[pl.ds(row0, NB1), :],
                    sidx.at[pl.ds(0, NB1), :])
    pltpu.sync_copy(dst_hbm.at[pl.ds(row0, NB1), :],
                    didx.at[pl.ds(0, NB1), :])

    @pl.when(ci == 0)
    def _():
      pltpu.sync_copy(src_hbm.at[pl.ds(row0 + NB1, NB0 - NB1), :],
                      sidx.at[pl.ds(NB1, NB0 - NB1), :])
      pltpu.sync_copy(dst_hbm.at[pl.ds(row0 + NB1, NB0 - NB1), :],
                      didx.at[pl.ds(NB1, NB0 - NB1), :])

    for i in range(128):
      zbuf[i, :] = jnp.zeros((16,), _f32)

    for c in range(nchunks):
      u = us[c]

      def gfire(j, bank, sem, b):
        pltpu.async_copy(u.at[sidx.at[j]],
                         bank.at[pl.ds(b * 128, 128), :], sem)

      def gwait(j, bank, sem, b):
        pltpu.make_async_copy(u.at[sidx.at[j]],
                              bank.at[pl.ds(b * 128, 128), :], sem).wait()

      def sfire(j, bank, b):
        pltpu.async_copy(bank.at[pl.ds(b * 128, 128), :],
                         acc.at[didx.at[j]], ssem, add=True)

      def swait(j, bank, b):
        pltpu.make_async_copy(bank.at[pl.ds(b * 128, 128), :],
                              acc.at[didx.at[j]], ssem).wait()

      def zrun(i, carry):
        pltpu.sync_copy(zbuf, acc.at[pl.ds(si * MSLICE + i * 128, 128), :])
        return carry

      lax.fori_loop(0, MSLICE // 128, zrun, 0)
      plsc.subcore_barrier()

      for b in range(K_BANK):
        gfire(b, rows_a, gsem_a, b)

      def pair(p, carry):
        g0 = 2 * p * K_BANK
        g1 = g0 + K_BANK
        g2 = g1 + K_BANK
        for b in range(K_BANK):
          gfire(g1 + b, rows_b, gsem_b, b)
        for b in range(K_BANK):
          gwait(g0 + b, rows_a, gsem_a, b)
        for b in range(K_BANK):
          sfire(g0 + b, rows_a, b)
        for b in range(K_BANK):
          swait(g0 + b, rows_a, b)

        @pl.when(p < npairs - 1)
        def _():
          for b in range(K_BANK):
            gfire(g2 + b, rows_a, gsem_a, b)

        for b in range(K_BANK):
          gwait(g1 + b, rows_b, gsem_b, b)
        for b in range(K_BANK):
          sfire(g1 + b, rows_b, b)
        for b in range(K_BANK):
          swait(g1 + b, rows_b, b)
        return carry

      lax.fori_loop(0, npairs, pair, 0)
      plsc.subcore_barrier()
      pltpu.sync_copy(acc.at[pl.ds(si * MSLICE, MSLICE), :],
                      outs[c].at[ci, pl.ds(si * MSLICE, MSLICE), :])
      if c + 1 < nchunks:
        plsc.subcore_barrier()

  def run(us, src2d, dst2d):
    k = pl.kernel(
        body,
        out_type=[jax.ShapeDtypeStruct((NC, NP, fc), _f32)
                  for _ in range(nchunks)],
        mesh=_mesh(),
        compiler_params=pltpu.CompilerParams(needs_layout_passes=False, use_tc_tiling_on_sc=False),
        scratch_types=[
            pltpu.VMEM((NB0, 128), jnp.int32),
            pltpu.VMEM((NB0, 128), jnp.int32),
            pltpu.VMEM((K_BANK * 128, fc), _f32),
            pltpu.VMEM((K_BANK * 128, fc), _f32),
            pltpu.VMEM((128, fc), _f32),
            pltpu.VMEM_SHARED((NP, fc), _f32),
            pltpu.SemaphoreType.DMA,
            pltpu.SemaphoreType.DMA,
            pltpu.SemaphoreType.DMA,
        ],
    )
    return k(*us, src2d, dst2d)

  return run


def _pool_body(xp_h, o1_h, o2_h, o3_h, b_hbm, zx_h, z1_h, z2_h, z3_h,
               px, p1, p2, p3,
               bidx, rx, r1, r2, r3, accx, acc1, acc2, acc3, sem):
  ci = lax.axis_index("c")
  si = lax.axis_index("s")
  wid = ci * NS + si
  g0 = si * GSLICE
  pltpu.sync_copy(zx_h, accx.at[pl.ds(g0, GSLICE), :])
  pltpu.sync_copy(z1_h, acc1.at[pl.ds(g0, GSLICE), :])
  pltpu.sync_copy(z2_h, acc2.at[pl.ds(g0, GSLICE), :])
  pltpu.sync_copy(z3_h, acc3.at[pl.ds(g0, GSLICE), :])
  plsc.subcore_barrier()

  for gi in range(2):
    grp = wid + gi * NT

    @pl.when(grp < NGRP)
    def _():
      pltpu.sync_copy(b_hbm.at[pl.ds(grp * 8, 8), :], bidx)

      def body(j, carry):
        n0 = grp * 1024 + j * 128
        pltpu.sync_copy(xp_h.at[pl.ds(n0, 128), :], rx)
        pltpu.sync_copy(o1_h.at[pl.ds(n0, 128), :], r1)
        pltpu.sync_copy(o2_h.at[pl.ds(n0, 128), :], r2)
        pltpu.sync_copy(o3_h.at[pl.ds(n0, 128), :], r3)
        idx = bidx.at[j]
        pltpu.sync_copy(rx, accx.at[idx], add=True)
        pltpu.sync_copy(r1, acc1.at[idx], add=True)
        pltpu.sync_copy(r2, acc2.at[idx], add=True)
        pltpu.sync_copy(r3, acc3.at[idx], add=True)
        return carry

      lax.fori_loop(0, 8, body, 0)

  plsc.subcore_barrier()
  pltpu.sync_copy(accx.at[pl.ds(g0, GSLICE), :],
                  px.at[ci, pl.ds(g0, GSLICE), :])
  pltpu.sync_copy(acc1.at[pl.ds(g0, GSLICE), :],
                  p1.at[ci, pl.ds(g0, GSLICE), :])
  pltpu.sync_copy(acc2.at[pl.ds(g0, GSLICE), :],
                  p2.at[ci, pl.ds(g0, GSLICE), :])
  pltpu.sync_copy(acc3.at[pl.ds(g0, GSLICE), :],
                  p3.at[ci, pl.ds(g0, GSLICE), :])


def _pool(xp, o1, o2, o3, b2d, zx, z1, z2, z3):
  k = pl.kernel(
      _pool_body,
      out_type=[
          jax.ShapeDtypeStruct((NC, GP, 112), _f32),
          jax.ShapeDtypeStruct((NC, GP, 64), _f32),
          jax.ShapeDtypeStruct((NC, GP, 32), _f32),
          jax.ShapeDtypeStruct((NC, GP, 16), _f32),
      ],
      mesh=_mesh(),
      compiler_params=pltpu.CompilerParams(needs_layout_passes=False, use_tc_tiling_on_sc=False),
      scratch_types=[
          pltpu.VMEM((8, 128), jnp.int32),
          pltpu.VMEM((128, 112), _f32),
          pltpu.VMEM((128, 64), _f32),
          pltpu.VMEM((128, 32), _f32),
          pltpu.VMEM((128, 16), _f32),
          pltpu.VMEM_SHARED((GP, 112), _f32),
          pltpu.VMEM_SHARED((GP, 64), _f32),
          pltpu.VMEM_SHARED((GP, 32), _f32),
          pltpu.VMEM_SHARED((GP, 16), _f32),
          pltpu.SemaphoreType.DMA,
      ],
  )
  return k(xp, o1, o2, o3, b2d, zx, z1, z2, z3)


def _m1_body(x_ref, w_ref, o_ref):
  o_ref[...] = jnp.dot(x_ref[...], w_ref[...], preferred_element_type=_f32)


def _m1(xp, w1tp):
  return pl.pallas_call(
      _m1_body,
      grid=(NRB,),
      in_specs=[
          pl.BlockSpec((RB, 112), lambda i: (i, 0)),
          pl.BlockSpec((112, 64), lambda i: (0, 0)),
      ],
      out_specs=pl.BlockSpec((RB, 64), lambda i: (i, 0)),
      out_shape=jax.ShapeDtypeStruct((NP, 64), _f32),
  )(xp, w1tp)


def _dsum_body(dp_ref, o_ref):
  o_ref[...] = jnp.sum(dp_ref[...], axis=0)


def _dsum(degp):
  lb = 4096
  return pl.pallas_call(
      _dsum_body,
      grid=(NP // 4096,),
      in_specs=[pl.BlockSpec((NT, lb), lambda i: (0, i))],
      out_specs=pl.BlockSpec((lb,), lambda i: (i,)),
      out_shape=jax.ShapeDtypeStruct((NP,), _f32),
  )(degp)


def _u1prep_body(dcol_ref, h1_ref, dinv_ref, dinv2_ref,
                 ua_ref, ub_ref, uc_ref, ud_ref):
  d = dcol_ref[...] + 1.0
  di = lax.rsqrt(d)
  dinv_ref[...] = di
  dinv2_ref[...] = 1.0 / d
  u = h1_ref[...] * di
  ua_ref[...] = u[:, 0:16]
  ub_ref[...] = u[:, 16:32]
  uc_ref[...] = u[:, 32:48]
  ud_ref[...] = u[:, 48:64]


def _u1prep(dcol, h1):
  return pl.pallas_call(
      _u1prep_body,
      grid=(NRB,),
      in_specs=[
          pl.BlockSpec((RB, 1), lambda i: (i, 0)),
          pl.BlockSpec((RB, 64), lambda i: (i, 0)),
      ],
      out_specs=[
          pl.BlockSpec((RB, 1), lambda i: (i, 0)),
          pl.BlockSpec((RB, 1), lambda i: (i, 0)),
          pl.BlockSpec((RB, 16), lambda i: (i, 0)),
          pl.BlockSpec((RB, 16), lambda i: (i, 0)),
          pl.BlockSpec((RB, 16), lambda i: (i, 0)),
          pl.BlockSpec((RB, 16), lambda i: (i, 0)),
      ],
      out_shape=[
          jax.ShapeDtypeStruct((NP, 1), _f32),
          jax.ShapeDtypeStruct((NP, 1), _f32),
          jax.ShapeDtypeStruct((NP, 16), _f32),
          jax.ShapeDtypeStruct((NP, 16), _f32),
          jax.ShapeDtypeStruct((NP, 16), _f32),
          jax.ShapeDtypeStruct((NP, 16), _f32),
      ],
  )(dcol, h1)


def _epi1_body(a0_ref, a1_ref, a2_ref, a3_ref, h1_ref, dinv_ref, dinv2_ref,
               b_ref, w_ref, o1_ref, h2_ref, u2a_ref, u2b_ref):
  agg = jnp.concatenate(
      [jnp.sum(a0_ref[...], axis=0), jnp.sum(a1_ref[...], axis=0),
       jnp.sum(a2_ref[...], axis=0), jnp.sum(a3_ref[...], axis=0)], axis=1)
  o1 = jnp.maximum(
      dinv_ref[...] * agg + dinv2_ref[...] * h1_ref[...] + b_ref[...], 0.0)
  o1_ref[...] = o1
  h2 = jnp.dot(o1, w_ref[...], preferred_element_type=_f32)
  h2_ref[...] = h2
  u2 = h2 * dinv_ref[...]
  u2a_ref[...] = u2[:, 0:16]
  u2b_ref[...] = u2[:, 16:32]


def _epi1(aggs, h1, dinv, dinv2, b1r, w2t):
  return pl.pallas_call(
      _epi1_body,
      grid=(NRB,),
      in_specs=[
          pl.BlockSpec((NC, RB, 16), lambda i: (0, i, 0)),
          pl.BlockSpec((NC, RB, 16), lambda i: (0, i, 0)),
          pl.BlockSpec((NC, RB, 16), lambda i: (0, i, 0)),
          pl.BlockSpec((NC, RB, 16), lambda i: (0, i, 0)),
          pl.BlockSpec((RB, 64), lambda i: (i, 0)),
          pl.BlockSpec((RB, 1), lambda i: (i, 0)),
          pl.BlockSpec((RB, 1), lambda i: (i, 0)),
          pl.BlockSpec((1, 64), lambda i: (0, 0)),
          pl.BlockSpec((64, 32), lambda i: (0, 0)),
      ],
      out_specs=[
          pl.BlockSpec((RB, 64), lambda i: (i, 0)),
          pl.BlockSpec((RB, 32), lambda i: (i, 0)),
          pl.BlockSpec((RB, 16), lambda i: (i, 0)),
          pl.BlockSpec((RB, 16), lambda i: (i, 0)),
      ],
      out_shape=[
          jax.ShapeDtypeStruct((NP, 64), _f32),
          jax.ShapeDtypeStruct((NP, 32), _f32),
          jax.ShapeDtypeStruct((NP, 16), _f32),
          jax.ShapeDtypeStruct((NP, 16), _f32),
      ],
  )(*aggs, h1, dinv, dinv2, b1r, w2t)


def _epi2_body(a0_ref, a1_ref, h2_ref, dinv_ref, dinv2_ref, b_ref, w_ref,
               o2_ref, h3_ref, u3_ref):
  agg = jnp.concatenate(
      [jnp.sum(a0_ref[...], axis=0), jnp.sum(a1_ref[...], axis=0)], axis=1)
  o2 = jnp.maximum(
      dinv_ref[...] * agg + dinv2_ref[...] * h2_ref[...] + b_ref[...], 0.0)
  o2_ref[...] = o2
  h3 = jnp.dot(o2, w_ref[...], preferred_element_type=_f32)
  h3_ref[...] = h3
  u3_ref[...] = h3 * dinv_ref[...]


def _epi2(aggs, h2, dinv, dinv2, b2r, w3t):
  return pl.pallas_call(
      _epi2_body,
      grid=(NRB,),
      in_specs=[
          pl.BlockSpec((NC, RB, 16), lambda i: (0, i, 0)),
          pl.BlockSpec((NC, RB, 16), lambda i: (0, i, 0)),
          pl.BlockSpec((RB, 32), lambda i: (i, 0)),
          pl.BlockSpec((RB, 1), lambda i: (i, 0)),
          pl.BlockSpec((RB, 1), lambda i: (i, 0)),
          pl.BlockSpec((1, 32), lambda i: (0, 0)),
          pl.BlockSpec((32, 16), lambda i: (0, 0)),
      ],
      out_specs=[
          pl.BlockSpec((RB, 32), lambda i: (i, 0)),
          pl.BlockSpec((RB, 16), lambda i: (i, 0)),
          pl.BlockSpec((RB, 16), lambda i: (i, 0)),
      ],
      out_shape=[
          jax.ShapeDtypeStruct((NP, 32), _f32),
          jax.ShapeDtypeStruct((NP, 16), _f32),
          jax.ShapeDtypeStruct((NP, 16), _f32),
      ],
  )(*aggs, h2, dinv, dinv2, b2r, w3t)


def _epi3_body(a_ref, h3_ref, dinv_ref, dinv2_ref, b_ref, o3_ref):
  agg = jnp.sum(a_ref[...], axis=0)
  o3_ref[...] = jnp.maximum(
      dinv_ref[...] * agg + dinv2_ref[...] * h3_ref[...] + b_ref[...], 0.0)


def _epi3(a, h3, dinv, dinv2, b3r):
  return pl.pallas_call(
      _epi3_body,
      grid=(NRB,),
      in_specs=[
          pl.BlockSpec((NC, RB, 16), lambda i: (0, i, 0)),
          pl.BlockSpec((RB, 16), lambda i: (i, 0)),
          pl.BlockSpec((RB, 1), lambda i: (i, 0)),
          pl.BlockSpec((RB, 1), lambda i: (i, 0)),
          pl.BlockSpec((1, 16), lambda i: (0, 0)),
      ],
      out_specs=pl.BlockSpec((RB, 16), lambda i: (i, 0)),
      out_shape=jax.ShapeDtypeStruct((NP, 16), _f32),
  )(a, h3, dinv, dinv2, b3r)


def _head_body(px_ref, p1_ref, p2_ref, p3_ref, bng_ref, bnb_ref, bnm_ref,
               bnv_ref, f1w_ref, f1b_ref, f2w_ref, f2b_ref, out_ref):
  px = px_ref[...]
  sx = px[0] + px[1]
  s1 = p1_ref[...][0] + p1_ref[...][1]
  s2 = p2_ref[...][0] + p2_ref[...][1]
  s3 = p3_ref[...][0] + p3_ref[...][1]
  cnt = jnp.maximum(sx[:, 111:112], 1.0)
  agg = jnp.concatenate([sx[:, :111], s1, s2, s3], axis=1) / cnt
  a = ((agg - bnm_ref[...]) * lax.rsqrt(bnv_ref[...] + 1e-5) * bng_ref[...]
       + bnb_ref[...])
  z = jnp.maximum(
      jnp.dot(a, f1w_ref[...], preferred_element_type=_f32) + f1b_ref[...],
      0.0)
  z = jnp.dot(z, f2w_ref[...], preferred_element_type=_f32) + f2b_ref[...]
  out_ref[...] = jax.nn.sigmoid(z[:G, :])


def _head(px, p1, p2, p3, bng, bnb, bnm, bnv, f1w, f1b, f2w, f2b):
  return pl.pallas_call(
      _head_body,
      out_shape=jax.ShapeDtypeStruct((G, 1), _f32),
  )(px, p1, p2, p3, bng, bnb, bnm, bnv, f1w, f1b, f2w, f2b)


def kernel(x, edge_index, batches, W1, b1, W2, b2, W3, b3,
           bn_gamma, bn_beta, bn_mean, bn_var, fc1_W, fc1_b, fc2_W, fc2_b):
  xp = jnp.pad(
      jnp.concatenate([x, jnp.ones((N, 1), _f32)], axis=1),
      ((0, NP - N), (0, 0)))
  src1 = jnp.pad(edge_index[0], (0, EP - E), constant_values=PAD_NODE)
  dst1 = jnp.pad(edge_index[1], (0, EP - E), constant_values=PAD_NODE)
  src2d = src1.reshape(EP // 128, 128)
  dst2d = dst1.reshape(EP // 128, 128)
  b2d = jnp.pad(batches, (0, NP - N), constant_values=G).reshape(NP // 128, 128)

  w1tp = jnp.pad(W1.T, ((0, 1), (0, 0)))
  w2t = W2.T
  w3t = W3.T
  b1r = b1.reshape(1, 64)
  b2r = b2.reshape(1, 32)
  b3r = b3.reshape(1, 16)
  f1w = fc1_W.T
  f1b = fc1_b.reshape(1, 64)
  f2w = fc2_W.T
  f2b = fc2_b.reshape(1, 1)
  bng = bn_gamma.reshape(1, 223)
  bnb = bn_beta.reshape(1, 223)
  bnm = bn_mean.reshape(1, 223)
  bnv = bn_var.reshape(1, 223)

  zeros_np = jnp.zeros((NP,), _f32)
  zgx = jnp.zeros((GSLICE, 112), _f32)
  zg1 = jnp.zeros((GSLICE, 64), _f32)
  zg2 = jnp.zeros((GSLICE, 32), _f32)
  zg3 = jnp.zeros((GSLICE, 16), _f32)

  h1 = _m1(xp, w1tp)
  degp = _deg(dst1, zeros_np)
  dsum = _dsum(degp.reshape(NT, NP))
  dinv, dinv2, u1a, u1b, u1c, u1d = _u1prep(dsum.reshape(NP, 1), h1)

  agg1 = _make_agg(16, 4)([u1a, u1b, u1c, u1d], src2d, dst2d)
  o1, h2, u2a, u2b = _epi1(agg1, h1, dinv, dinv2, b1r, w2t)

  agg2 = _make_agg(16, 2)([u2a, u2b], src2d, dst2d)
  o2, h3, u3 = _epi2(agg2, h2, dinv, dinv2, b2r, w3t)

  agg3 = _make_agg(16, 1)([u3], src2d, dst2d)
  o3 = _epi3(agg3[0], h3, dinv, dinv2, b3r)

  px, p1, p2, p3 = _pool(xp, o1, o2, o3, b2d, zgx, zg1, zg2, zg3)
  return _head(px, p1, p2, p3, bng, bnb, bnm, bnv, f1w, f1b, f2w, f2b)


# speedup vs baseline: 1.1079x; 1.0381x over previous
import functools

import jax
import jax.numpy as jnp
from jax import lax
from jax.experimental import pallas as pl
from jax.experimental.pallas import tpu as pltpu
from jax.experimental.pallas import tpu_sc as plsc

N = 44400
E = 710400
G = 400
F0 = 111

NC = 2
NS = 16
NT = NC * NS

NP = 45056
EP = 720896
GP = 512
PAD_NODE = N

NB_E = EP // 128 // NT
MSLICE = NP // NS
GSLICE = GP // NS
NGRP = NP // 1024

RB = 2816
NRB = NP // RB

_f32 = jnp.float32


def _mesh():
  return plsc.VectorSubcoreMesh(core_axis_name="c", subcore_axis_name="s")


def _deg_body(dst_hbm, zeros_hbm, out_hbm, acc, didx):
  ci = lax.axis_index("c")
  si = lax.axis_index("s")
  wid = ci * NS + si
  ept = EP // NT
  pltpu.sync_copy(zeros_hbm, acc)
  pltpu.sync_copy(dst_hbm.at[pl.ds(wid * ept, ept)], didx)
  ones = jnp.ones((16,), _f32)

  def body(j, carry):
    idx = didx[pl.ds(j * 16, 16)]
    plsc.addupdate_scatter(acc, [idx], ones)
    return carry

  lax.fori_loop(0, ept // 16, body, 0)
  pltpu.sync_copy(acc, out_hbm.at[pl.ds(wid * NP, NP)])


def _deg(dst1d, zeros_np):
  k = pl.kernel(
      _deg_body,
      out_type=jax.ShapeDtypeStruct((NT * NP,), _f32),
      mesh=_mesh(),
      compiler_params=pltpu.CompilerParams(needs_layout_passes=False, use_tc_tiling_on_sc=False),
      scratch_types=[
          pltpu.VMEM((NP,), _f32),
          pltpu.VMEM((EP // NT,), jnp.int32),
      ],
  )
  return k(dst1d, zeros_np)


K_BANK = 8
NBH = 176
NPAIRS = NBH // (2 * K_BANK)


def _make_agg(fc, chunks0, chunks1):
  nchunks = len(chunks0) + len(chunks1)

  def body(*refs):
    us = refs[:nchunks]
    src_hbm, dst_hbm = refs[nchunks:nchunks + 2]
    outs = refs[nchunks + 2:2 * nchunks + 2]
    (sidx, didx, rows_a, rows_b, zbuf, acc,
     gsem_a, gsem_b, ssem) = refs[2 * nchunks + 2:]
    ci = lax.axis_index("c")
    si = lax.axis_index("s")

    for i in range(128):
      zbuf[i, :] = jnp.zeros((16,), _f32)

    def run_chunk(c):
      u = us[c]

      def gfire(j, bank, sem, b):
        pltpu.async_copy(u.at[sidx.at[j]],
                         bank.at[pl.ds(b * 128, 128), :], sem)

      def gwait(j, bank, sem, b):
        pltpu.make_async_copy(u.at[sidx.at[j]],
                              bank.at[pl.ds(b * 128, 128), :], sem).wait()

      def sfire(j, bank, b):
        pltpu.async_copy(bank.at[pl.ds(b * 128, 128), :],
                         acc.at[didx.at[j]], ssem, add=True)

      def swait(j, bank, b):
        pltpu.make_async_copy(bank.at[pl.ds(b * 128, 128), :],
                              acc.at[didx.at[j]], ssem).wait()

      def zrun(i, carry):
        pltpu.sync_copy(zbuf, acc.at[pl.ds(si * MSLICE + i * 128, 128), :])
        return carry

      lax.fori_loop(0, MSLICE // 128, zrun, 0)
      plsc.subcore_barrier()

      for half in range(2):
        row0 = si * (2 * NBH) + half * NBH
        pltpu.sync_copy(src_hbm.at[pl.ds(row0, NBH), :], sidx)
        pltpu.sync_copy(dst_hbm.at[pl.ds(row0, NBH), :], didx)

        for b in range(K_BANK):
          gfire(b, rows_a, gsem_a, b)

        def pair(p, carry):
          g0 = 2 * p * K_BANK
          g1 = g0 + K_BANK
          g2 = g1 + K_BANK
          for b in range(K_BANK):
            gfire(g1 + b, rows_b, gsem_b, b)
          for b in range(K_BANK):
            gwait(g0 + b, rows_a, gsem_a, b)
          for b in range(K_BANK):
            sfire(g0 + b, rows_a, b)
          for b in range(K_BANK):
            swait(g0 + b, rows_a, b)

          @pl.when(p < NPAIRS - 1)
          def _():
            for b in range(K_BANK):
              gfire(g2 + b, rows_a, gsem_a, b)

          for b in range(K_BANK):
            gwait(g1 + b, rows_b, gsem_b, b)
          for b in range(K_BANK):
            sfire(g1 + b, rows_b, b)
          for b in range(K_BANK):
            swait(g1 + b, rows_b, b)
          return carry

        lax.fori_loop(0, NPAIRS, pair, 0)

      plsc.subcore_barrier()
      pltpu.sync_copy(acc.at[pl.ds(si * MSLICE, MSLICE), :],
                      outs[c].at[pl.ds(si * MSLICE, MSLICE), :])

    @pl.when(ci == 0)
    def _():
      for k, c in enumerate(chunks0):
        run_chunk(c)
        if k + 1 < len(chunks0):
          plsc.subcore_barrier()

    @pl.when(ci == 1)
    def _():
      for k, c in enumerate(chunks1):
        run_chunk(c)
        if k + 1 < len(chunks1):
          plsc.subcore_barrier()

  def run(us, src2d, dst2d):
    k = pl.kernel(
        body,
        out_type=[jax.ShapeDtypeStruct((NP, fc), _f32)
                  for _ in range(nchunks)],
        mesh=_mesh(),
        compiler_params=pltpu.CompilerParams(needs_layout_passes=False, use_tc_tiling_on_sc=False),
        scratch_types=[
            pltpu.VMEM((NBH, 128), jnp.int32),
            pltpu.VMEM((NBH, 128), jnp.int32),
            pltpu.VMEM((K_BANK * 128, fc), _f32),
            pltpu.VMEM((K_BANK * 128, fc), _f32),
            pltpu.VMEM((128, fc), _f32),
            pltpu.VMEM_SHARED((NP, fc), _f32),
            pltpu.SemaphoreType.DMA,
            pltpu.SemaphoreType.DMA,
            pltpu.SemaphoreType.DMA,
        ],
    )
    return k(*us, src2d, dst2d)

  return run


def _pool_body(xp_h, o1_h, o2_h, o3_h, b_hbm, zx_h, z1_h, z2_h, z3_h,
               px, p1, p2, p3,
               bidx, rx, r1, r2, r3, accx, acc1, acc2, acc3, sem):
  ci = lax.axis_index("c")
  si = lax.axis_index("s")
  wid = ci * NS + si
  g0 = si * GSLICE
  pltpu.sync_copy(zx_h, accx.at[pl.ds(g0, GSLICE), :])
  pltpu.sync_copy(z1_h, acc1.at[pl.ds(g0, GSLICE), :])
  pltpu.sync_copy(z2_h, acc2.at[pl.ds(g0, GSLICE), :])
  pltpu.sync_copy(z3_h, acc3.at[pl.ds(g0, GSLICE), :])
  plsc.subcore_barrier()

  for gi in range(2):
    grp = wid + gi * NT

    @pl.when(grp < NGRP)
    def _():
      pltpu.sync_copy(b_hbm.at[pl.ds(grp * 8, 8), :], bidx)

      def body(j, carry):
        n0 = grp * 1024 + j * 128
        pltpu.sync_copy(xp_h.at[pl.ds(n0, 128), :], rx)
        pltpu.sync_copy(o1_h.at[pl.ds(n0, 128), :], r1)
        pltpu.sync_copy(o2_h.at[pl.ds(n0, 128), :], r2)
        pltpu.sync_copy(o3_h.at[pl.ds(n0, 128), :], r3)
        idx = bidx.at[j]
        pltpu.sync_copy(rx, accx.at[idx], add=True)
        pltpu.sync_copy(r1, acc1.at[idx], add=True)
        pltpu.sync_copy(r2, acc2.at[idx], add=True)
        pltpu.sync_copy(r3, acc3.at[idx], add=True)
        return carry

      lax.fori_loop(0, 8, body, 0)

  plsc.subcore_barrier()
  pltpu.sync_copy(accx.at[pl.ds(g0, GSLICE), :],
                  px.at[ci, pl.ds(g0, GSLICE), :])
  pltpu.sync_copy(acc1.at[pl.ds(g0, GSLICE), :],
                  p1.at[ci, pl.ds(g0, GSLICE), :])
  pltpu.sync_copy(acc2.at[pl.ds(g0, GSLICE), :],
                  p2.at[ci, pl.ds(g0, GSLICE), :])
  pltpu.sync_copy(acc3.at[pl.ds(g0, GSLICE), :],
                  p3.at[ci, pl.ds(g0, GSLICE), :])


def _pool(xp, o1, o2, o3, b2d, zx, z1, z2, z3):
  k = pl.kernel(
      _pool_body,
      out_type=[
          jax.ShapeDtypeStruct((NC, GP, 112), _f32),
          jax.ShapeDtypeStruct((NC, GP, 64), _f32),
          jax.ShapeDtypeStruct((NC, GP, 32), _f32),
          jax.ShapeDtypeStruct((NC, GP, 16), _f32),
      ],
      mesh=_mesh(),
      compiler_params=pltpu.CompilerParams(needs_layout_passes=False, use_tc_tiling_on_sc=False),
      scratch_types=[
          pltpu.VMEM((8, 128), jnp.int32),
          pltpu.VMEM((128, 112), _f32),
          pltpu.VMEM((128, 64), _f32),
          pltpu.VMEM((128, 32), _f32),
          pltpu.VMEM((128, 16), _f32),
          pltpu.VMEM_SHARED((GP, 112), _f32),
          pltpu.VMEM_SHARED((GP, 64), _f32),
          pltpu.VMEM_SHARED((GP, 32), _f32),
          pltpu.VMEM_SHARED((GP, 16), _f32),
          pltpu.SemaphoreType.DMA,
      ],
  )
  return k(xp, o1, o2, o3, b2d, zx, z1, z2, z3)


def _m1_body(x_ref, w_ref, o_ref):
  o_ref[...] = jnp.dot(x_ref[...], w_ref[...], preferred_element_type=_f32)


def _m1(xp, w1tp):
  return pl.pallas_call(
      _m1_body,
      grid=(NRB,),
      in_specs=[
          pl.BlockSpec((RB, 112), lambda i: (i, 0)),
          pl.BlockSpec((112, 64), lambda i: (0, 0)),
      ],
      out_specs=pl.BlockSpec((RB, 64), lambda i: (i, 0)),
      out_shape=jax.ShapeDtypeStruct((NP, 64), _f32),
  )(xp, w1tp)


def _dsum_body(dp_ref, o_ref):
  o_ref[...] = jnp.sum(dp_ref[...], axis=0)


def _dsum(degp):
  lb = 4096
  return pl.pallas_call(
      _dsum_body,
      grid=(NP // 4096,),
      in_specs=[pl.BlockSpec((NT, lb), lambda i: (0, i))],
      out_specs=pl.BlockSpec((lb,), lambda i: (i,)),
      out_shape=jax.ShapeDtypeStruct((NP,), _f32),
  )(degp)


def _u1prep_body(dcol_ref, h1_ref, dinv_ref, dinv2_ref,
                 ua_ref, ub_ref, uc_ref, ud_ref):
  d = dcol_ref[...] + 1.0
  di = lax.rsqrt(d)
  dinv_ref[...] = di
  dinv2_ref[...] = 1.0 / d
  u = h1_ref[...] * di
  ua_ref[...] = u[:, 0:16]
  ub_ref[...] = u[:, 16:32]
  uc_ref[...] = u[:, 32:48]
  ud_ref[...] = u[:, 48:64]


def _u1prep(dcol, h1):
  return pl.pallas_call(
      _u1prep_body,
      grid=(NRB,),
      in_specs=[
          pl.BlockSpec((RB, 1), lambda i: (i, 0)),
          pl.BlockSpec((RB, 64), lambda i: (i, 0)),
      ],
      out_specs=[
          pl.BlockSpec((RB, 1), lambda i: (i, 0)),
          pl.BlockSpec((RB, 1), lambda i: (i, 0)),
          pl.BlockSpec((RB, 16), lambda i: (i, 0)),
          pl.BlockSpec((RB, 16), lambda i: (i, 0)),
          pl.BlockSpec((RB, 16), lambda i: (i, 0)),
          pl.BlockSpec((RB, 16), lambda i: (i, 0)),
      ],
      out_shape=[
          jax.ShapeDtypeStruct((NP, 1), _f32),
          jax.ShapeDtypeStruct((NP, 1), _f32),
          jax.ShapeDtypeStruct((NP, 16), _f32),
          jax.ShapeDtypeStruct((NP, 16), _f32),
          jax.ShapeDtypeStruct((NP, 16), _f32),
          jax.ShapeDtypeStruct((NP, 16), _f32),
      ],
  )(dcol, h1)


def _epi1_body(a0_ref, a1_ref, a2_ref, a3_ref, h1_ref, dinv_ref, dinv2_ref,
               b_ref, w_ref, o1_ref, h2_ref, u2a_ref, u2b_ref):
  agg = jnp.concatenate(
      [a0_ref[...], a1_ref[...], a2_ref[...], a3_ref[...]], axis=1)
  o1 = jnp.maximum(
      dinv_ref[...] * agg + dinv2_ref[...] * h1_ref[...] + b_ref[...], 0.0)
  o1_ref[...] = o1
  h2 = jnp.dot(o1, w_ref[...], preferred_element_type=_f32)
  h2_ref[...] = h2
  u2 = h2 * dinv_ref[...]
  u2a_ref[...] = u2[:, 0:16]
  u2b_ref[...] = u2[:, 16:32]


def _epi1(aggs, h1, dinv, dinv2, b1r, w2t):
  return pl.pallas_call(
      _epi1_body,
      grid=(NRB,),
      in_specs=[
          pl.BlockSpec((RB, 16), lambda i: (i, 0)),
          pl.BlockSpec((RB, 16), lambda i: (i, 0)),
          pl.BlockSpec((RB, 16), lambda i: (i, 0)),
          pl.BlockSpec((RB, 16), lambda i: (i, 0)),
          pl.BlockSpec((RB, 64), lambda i: (i, 0)),
          pl.BlockSpec((RB, 1), lambda i: (i, 0)),
          pl.BlockSpec((RB, 1), lambda i: (i, 0)),
          pl.BlockSpec((1, 64), lambda i: (0, 0)),
          pl.BlockSpec((64, 32), lambda i: (0, 0)),
      ],
      out_specs=[
          pl.BlockSpec((RB, 64), lambda i: (i, 0)),
          pl.BlockSpec((RB, 32), lambda i: (i, 0)),
          pl.BlockSpec((RB, 16), lambda i: (i, 0)),
          pl.BlockSpec((RB, 16), lambda i: (i, 0)),
      ],
      out_shape=[
          jax.ShapeDtypeStruct((NP, 64), _f32),
          jax.ShapeDtypeStruct((NP, 32), _f32),
          jax.ShapeDtypeStruct((NP, 16), _f32),
          jax.ShapeDtypeStruct((NP, 16), _f32),
      ],
  )(*aggs, h1, dinv, dinv2, b1r, w2t)


def _epi2_body(a0_ref, a1_ref, h2_ref, dinv_ref, dinv2_ref, b_ref, w_ref,
               o2_ref, h3_ref, u3_ref):
  agg = jnp.concatenate([a0_ref[...], a1_ref[...]], axis=1)
  o2 = jnp.maximum(
      dinv_ref[...] * agg + dinv2_ref[...] * h2_ref[...] + b_ref[...], 0.0)
  o2_ref[...] = o2
  h3 = jnp.dot(o2, w_ref[...], preferred_element_type=_f32)
  h3_ref[...] = h3
  u3_ref[...] = h3 * dinv_ref[...]


def _epi2(aggs, h2, dinv, dinv2, b2r, w3t):
  return pl.pallas_call(
      _epi2_body,
      grid=(NRB,),
      in_specs=[
          pl.BlockSpec((RB, 16), lambda i: (i, 0)),
          pl.BlockSpec((RB, 16), lambda i: (i, 0)),
          pl.BlockSpec((RB, 32), lambda i: (i, 0)),
          pl.BlockSpec((RB, 1), lambda i: (i, 0)),
          pl.BlockSpec((RB, 1), lambda i: (i, 0)),
          pl.BlockSpec((1, 32), lambda i: (0, 0)),
          pl.BlockSpec((32, 16), lambda i: (0, 0)),
      ],
      out_specs=[
          pl.BlockSpec((RB, 32), lambda i: (i, 0)),
          pl.BlockSpec((RB, 16), lambda i: (i, 0)),
          pl.BlockSpec((RB, 16), lambda i: (i, 0)),
      ],
      out_shape=[
          jax.ShapeDtypeStruct((NP, 32), _f32),
          jax.ShapeDtypeStruct((NP, 16), _f32),
          jax.ShapeDtypeStruct((NP, 16), _f32),
      ],
  )(*aggs, h2, dinv, dinv2, b2r, w3t)


def _epi3_body(a_ref, h3_ref, dinv_ref, dinv2_ref, b_ref, o3_ref):
  agg = a_ref[...]
  o3_ref[...] = jnp.maximum(
      dinv_ref[...] * agg + dinv2_ref[...] * h3_ref[...] + b_ref[...], 0.0)


def _epi3(a, h3, dinv, dinv2, b3r):
  return pl.pallas_call(
      _epi3_body,
      grid=(NRB,),
      in_specs=[
          pl.BlockSpec((RB, 16), lambda i: (i, 0)),
          pl.BlockSpec((RB, 16), lambda i: (i, 0)),
          pl.BlockSpec((RB, 1), lambda i: (i, 0)),
          pl.BlockSpec((RB, 1), lambda i: (i, 0)),
          pl.BlockSpec((1, 16), lambda i: (0, 0)),
      ],
      out_specs=pl.BlockSpec((RB, 16), lambda i: (i, 0)),
      out_shape=jax.ShapeDtypeStruct((NP, 16), _f32),
  )(a, h3, dinv, dinv2, b3r)


def _head_body(px_ref, p1_ref, p2_ref, p3_ref, bng_ref, bnb_ref, bnm_ref,
               bnv_ref, f1w_ref, f1b_ref, f2w_ref, f2b_ref, out_ref):
  px = px_ref[...]
  sx = px[0] + px[1]
  s1 = p1_ref[...][0] + p1_ref[...][1]
  s2 = p2_ref[...][0] + p2_ref[...][1]
  s3 = p3_ref[...][0] + p3_ref[...][1]
  cnt = jnp.maximum(sx[:, 111:112], 1.0)
  agg = jnp.concatenate([sx[:, :111], s1, s2, s3], axis=1) / cnt
  a = ((agg - bnm_ref[...]) * lax.rsqrt(bnv_ref[...] + 1e-5) * bng_ref[...]
       + bnb_ref[...])
  z = jnp.maximum(
      jnp.dot(a, f1w_ref[...], preferred_element_type=_f32) + f1b_ref[...],
      0.0)
  z = jnp.dot(z, f2w_ref[...], preferred_element_type=_f32) + f2b_ref[...]
  out_ref[...] = jax.nn.sigmoid(z[:G, :])


def _head(px, p1, p2, p3, bng, bnb, bnm, bnv, f1w, f1b, f2w, f2b):
  return pl.pallas_call(
      _head_body,
      out_shape=jax.ShapeDtypeStruct((G, 1), _f32),
  )(px, p1, p2, p3, bng, bnb, bnm, bnv, f1w, f1b, f2w, f2b)


def kernel(x, edge_index, batches, W1, b1, W2, b2, W3, b3,
           bn_gamma, bn_beta, bn_mean, bn_var, fc1_W, fc1_b, fc2_W, fc2_b):
  xp = jnp.pad(
      jnp.concatenate([x, jnp.ones((N, 1), _f32)], axis=1),
      ((0, NP - N), (0, 0)))
  src1 = jnp.pad(edge_index[0], (0, EP - E), constant_values=PAD_NODE)
  dst1 = jnp.pad(edge_index[1], (0, EP - E), constant_values=PAD_NODE)
  src2d = src1.reshape(EP // 128, 128)
  dst2d = dst1.reshape(EP // 128, 128)
  b2d = jnp.pad(batches, (0, NP - N), constant_values=G).reshape(NP // 128, 128)

  w1tp = jnp.pad(W1.T, ((0, 1), (0, 0)))
  w2t = W2.T
  w3t = W3.T
  b1r = b1.reshape(1, 64)
  b2r = b2.reshape(1, 32)
  b3r = b3.reshape(1, 16)
  f1w = fc1_W.T
  f1b = fc1_b.reshape(1, 64)
  f2w = fc2_W.T
  f2b = fc2_b.reshape(1, 1)
  bng = bn_gamma.reshape(1, 223)
  bnb = bn_beta.reshape(1, 223)
  bnm = bn_mean.reshape(1, 223)
  bnv = bn_var.reshape(1, 223)

  zeros_np = jnp.zeros((NP,), _f32)
  zgx = jnp.zeros((GSLICE, 112), _f32)
  zg1 = jnp.zeros((GSLICE, 64), _f32)
  zg2 = jnp.zeros((GSLICE, 32), _f32)
  zg3 = jnp.zeros((GSLICE, 16), _f32)

  h1 = _m1(xp, w1tp)
  degp = _deg(dst1, zeros_np)
  dsum = _dsum(degp.reshape(NT, NP))
  dinv, dinv2, u1a, u1b, u1c, u1d = _u1prep(dsum.reshape(NP, 1), h1)

  agg1 = _make_agg(16, [0, 1, 2], [3])([u1a, u1b, u1c, u1d], src2d, dst2d)
  o1, h2, u2a, u2b = _epi1(agg1, h1, dinv, dinv2, b1r, w2t)

  agg2 = _make_agg(16, [0, 1], [])([u2a, u2b], src2d, dst2d)
  o2, h3, u3 = _epi2(agg2, h2, dinv, dinv2, b2r, w3t)

  agg3 = _make_agg(16, [0], [])([u3], src2d, dst2d)
  o3 = _epi3(agg3[0], h3, dinv, dinv2, b3r)

  px, p1, p2, p3 = _pool(xp, o1, o2, o3, b2d, zgx, zg1, zg2, zg3)
  return _head(px, p1, p2, p3, bng, bnb, bnm, bnv, f1w, f1b, f2w, f2b)


# speedup vs baseline: 1.6012x; 1.4452x over previous
import functools

import jax
import jax.numpy as jnp
from jax import lax
from jax.experimental import pallas as pl
from jax.experimental.pallas import tpu as pltpu
from jax.experimental.pallas import tpu_sc as plsc

N = 44400
E = 710400
G = 400
F0 = 111

NC = 2
NS = 16
NT = NC * NS

NP = 45056
EP = 720896
GP = 512
PAD_NODE = N

NB_E = EP // 128 // NT
MSLICE = NP // NS
GSLICE = GP // NS
NGRP = NP // 1024

RB = 2816
NRB = NP // RB

_f32 = jnp.float32


def _mesh():
  return plsc.VectorSubcoreMesh(core_axis_name="c", subcore_axis_name="s")


def _deg_body(dst_hbm, zeros_hbm, out_hbm, acc, didx):
  ci = lax.axis_index("c")
  si = lax.axis_index("s")
  wid = ci * NS + si
  ept = EP // NT
  pltpu.sync_copy(zeros_hbm, acc)
  pltpu.sync_copy(dst_hbm.at[pl.ds(wid * ept, ept)], didx)
  ones = jnp.ones((16,), _f32)

  def body(j, carry):
    idx = didx[pl.ds(j * 16, 16)]
    plsc.addupdate_scatter(acc, [idx], ones)
    return carry

  lax.fori_loop(0, ept // 16, body, 0)
  pltpu.sync_copy(acc, out_hbm.at[pl.ds(wid * NP, NP)])


def _deg(dst1d, zeros_np):
  k = pl.kernel(
      _deg_body,
      out_type=jax.ShapeDtypeStruct((NT * NP,), _f32),
      mesh=_mesh(),
      compiler_params=pltpu.CompilerParams(needs_layout_passes=False, use_tc_tiling_on_sc=False),
      scratch_types=[
          pltpu.VMEM((NP,), _f32),
          pltpu.VMEM((EP // NT,), jnp.int32),
      ],
  )
  return k(dst1d, zeros_np)


K_BANK = 8
NBH = 176
NPAIRS = NBH // (2 * K_BANK)


def _make_agg(fc, chunks0, chunks1):
  nchunks = len(chunks0) + len(chunks1)

  def body(*refs):
    us = refs[:nchunks]
    src_hbm, dst_hbm = refs[nchunks:nchunks + 2]
    outs = refs[nchunks + 2:2 * nchunks + 2]
    (sidx, didx, rows_a, rows_b, zbuf, acc,
     gsem_a, gsem_b, ssem) = refs[2 * nchunks + 2:]
    ci = lax.axis_index("c")
    si = lax.axis_index("s")

    for i in range(128):
      zbuf[i, :] = jnp.zeros((16,), _f32)

    def run_chunk(c):
      u = us[c]

      def gfire(j, bank, sem, b):
        pltpu.async_copy(u.at[sidx.at[j]],
                         bank.at[pl.ds(b * 128, 128), :], sem)

      def gwait(j, bank, sem, b):
        pltpu.make_async_copy(u.at[sidx.at[j]],
                              bank.at[pl.ds(b * 128, 128), :], sem).wait()

      def sfire(j, bank, b):
        pltpu.async_copy(bank.at[pl.ds(b * 128, 128), :],
                         acc.at[didx.at[j]], ssem, add=True)

      def swait(j, bank, b):
        pltpu.make_async_copy(bank.at[pl.ds(b * 128, 128), :],
                              acc.at[didx.at[j]], ssem).wait()

      def zrun(i, carry):
        pltpu.sync_copy(zbuf, acc.at[pl.ds(si * MSLICE + i * 128, 128), :])
        return carry

      lax.fori_loop(0, MSLICE // 128, zrun, 0)
      plsc.subcore_barrier()

      for half in range(2):
        row0 = si * (2 * NBH) + half * NBH
        pltpu.sync_copy(src_hbm.at[pl.ds(row0, NBH), :], sidx)
        pltpu.sync_copy(dst_hbm.at[pl.ds(row0, NBH), :], didx)

        for b in range(K_BANK):
          gfire(b, rows_a, gsem_a, b)

        def pair(p, carry):
          g0 = 2 * p * K_BANK
          g1 = g0 + K_BANK
          g2 = g1 + K_BANK
          for b in range(K_BANK):
            gfire(g1 + b, rows_b, gsem_b, b)
          for b in range(K_BANK):
            gwait(g0 + b, rows_a, gsem_a, b)
          for b in range(K_BANK):
            sfire(g0 + b, rows_a, b)
          for b in range(K_BANK):
            swait(g0 + b, rows_a, b)

          @pl.when(p < NPAIRS - 1)
          def _():
            for b in range(K_BANK):
              gfire(g2 + b, rows_a, gsem_a, b)

          for b in range(K_BANK):
            gwait(g1 + b, rows_b, gsem_b, b)
          for b in range(K_BANK):
            sfire(g1 + b, rows_b, b)
          for b in range(K_BANK):
            swait(g1 + b, rows_b, b)
          return carry

        lax.fori_loop(0, NPAIRS, pair, 0)

      plsc.subcore_barrier()
      pltpu.sync_copy(acc.at[pl.ds(si * MSLICE, MSLICE), :],
                      outs[c].at[pl.ds(si * MSLICE, MSLICE), :])

    @pl.when(ci == 0)
    def _():
      for k, c in enumerate(chunks0):
        run_chunk(c)
        if k + 1 < len(chunks0):
          plsc.subcore_barrier()

    @pl.when(ci == 1)
    def _():
      for k, c in enumerate(chunks1):
        run_chunk(c)
        if k + 1 < len(chunks1):
          plsc.subcore_barrier()

  def run(us, src2d, dst2d):
    k = pl.kernel(
        body,
        out_type=[jax.ShapeDtypeStruct((NP, fc), _f32)
                  for _ in range(nchunks)],
        mesh=_mesh(),
        compiler_params=pltpu.CompilerParams(needs_layout_passes=False, use_tc_tiling_on_sc=False),
        scratch_types=[
            pltpu.VMEM((NBH, 128), jnp.int32),
            pltpu.VMEM((NBH, 128), jnp.int32),
            pltpu.VMEM((K_BANK * 128, fc), _f32),
            pltpu.VMEM((K_BANK * 128, fc), _f32),
            pltpu.VMEM((128, fc), _f32),
            pltpu.VMEM_SHARED((NP, fc), _f32),
            pltpu.SemaphoreType.DMA,
            pltpu.SemaphoreType.DMA,
            pltpu.SemaphoreType.DMA,
        ],
    )
    return k(*us, src2d, dst2d)

  return run


def _pool_body(xp_h, o1_h, o2_h, o3_h, b_hbm, zx_h, z1_h, z2_h, z3_h,
               px, p1, p2, p3,
               bidx, rx, r1, r2, r3, accx, acc1, acc2, acc3, sem):
  ci = lax.axis_index("c")
  si = lax.axis_index("s")
  wid = ci * NS + si
  g0 = si * GSLICE
  pltpu.sync_copy(zx_h, accx.at[pl.ds(g0, GSLICE), :])
  pltpu.sync_copy(z1_h, acc1.at[pl.ds(g0, GSLICE), :])
  pltpu.sync_copy(z2_h, acc2.at[pl.ds(g0, GSLICE), :])
  pltpu.sync_copy(z3_h, acc3.at[pl.ds(g0, GSLICE), :])
  plsc.subcore_barrier()

  for gi in range(2):
    grp = wid + gi * NT

    @pl.when(grp < NGRP)
    def _():
      pltpu.sync_copy(b_hbm.at[pl.ds(grp * 8, 8), :], bidx)

      def body(j, carry):
        n0 = grp * 1024 + j * 128
        pltpu.sync_copy(xp_h.at[pl.ds(n0, 128), :], rx)
        pltpu.sync_copy(o1_h.at[pl.ds(n0, 128), :], r1)
        pltpu.sync_copy(o2_h.at[pl.ds(n0, 128), :], r2)
        pltpu.sync_copy(o3_h.at[pl.ds(n0, 128), :], r3)
        idx = bidx.at[j]
        pltpu.sync_copy(rx, accx.at[idx], add=True)
        pltpu.sync_copy(r1, acc1.at[idx], add=True)
        pltpu.sync_copy(r2, acc2.at[idx], add=True)
        pltpu.sync_copy(r3, acc3.at[idx], add=True)
        return carry

      lax.fori_loop(0, 8, body, 0)

  plsc.subcore_barrier()
  pltpu.sync_copy(accx.at[pl.ds(g0, GSLICE), :],
                  px.at[ci, pl.ds(g0, GSLICE), :])
  pltpu.sync_copy(acc1.at[pl.ds(g0, GSLICE), :],
                  p1.at[ci, pl.ds(g0, GSLICE), :])
  pltpu.sync_copy(acc2.at[pl.ds(g0, GSLICE), :],
                  p2.at[ci, pl.ds(g0, GSLICE), :])
  pltpu.sync_copy(acc3.at[pl.ds(g0, GSLICE), :],
                  p3.at[ci, pl.ds(g0, GSLICE), :])


def _pool(xp, o1, o2, o3, b2d, zx, z1, z2, z3):
  k = pl.kernel(
      _pool_body,
      out_type=[
          jax.ShapeDtypeStruct((NC, GP, 112), _f32),
          jax.ShapeDtypeStruct((NC, GP, 64), _f32),
          jax.ShapeDtypeStruct((NC, GP, 32), _f32),
          jax.ShapeDtypeStruct((NC, GP, 16), _f32),
      ],
      mesh=_mesh(),
      compiler_params=pltpu.CompilerParams(needs_layout_passes=False, use_tc_tiling_on_sc=False),
      scratch_types=[
          pltpu.VMEM((8, 128), jnp.int32),
          pltpu.VMEM((128, 112), _f32),
          pltpu.VMEM((128, 64), _f32),
          pltpu.VMEM((128, 32), _f32),
          pltpu.VMEM((128, 16), _f32),
          pltpu.VMEM_SHARED((GP, 112), _f32),
          pltpu.VMEM_SHARED((GP, 64), _f32),
          pltpu.VMEM_SHARED((GP, 32), _f32),
          pltpu.VMEM_SHARED((GP, 16), _f32),
          pltpu.SemaphoreType.DMA,
      ],
  )
  return k(xp, o1, o2, o3, b2d, zx, z1, z2, z3)


def _m1_body(x_ref, w_ref, o_ref):
  o_ref[...] = jnp.dot(x_ref[...], w_ref[...], preferred_element_type=_f32)


def _m1(xp, w1tp):
  return pl.pallas_call(
      _m1_body,
      grid=(NRB,),
      in_specs=[
          pl.BlockSpec((RB, 112), lambda i: (i, 0)),
          pl.BlockSpec((112, 64), lambda i: (0, 0)),
      ],
      out_specs=pl.BlockSpec((RB, 64), lambda i: (i, 0)),
      out_shape=jax.ShapeDtypeStruct((NP, 64), _f32),
  )(xp, w1tp)


def _dsum_body(dp_ref, o_ref):
  o_ref[...] = jnp.sum(dp_ref[...], axis=0)


def _dsum(degp):
  lb = 4096
  return pl.pallas_call(
      _dsum_body,
      grid=(NP // 4096,),
      in_specs=[pl.BlockSpec((NT, lb), lambda i: (0, i))],
      out_specs=pl.BlockSpec((lb,), lambda i: (i,)),
      out_shape=jax.ShapeDtypeStruct((NP,), _f32),
  )(degp)


def _u1prep_body(dcol_ref, h1_ref, dinv_ref, dinv2_ref,
                 ua_ref, ub_ref, uc_ref, ud_ref):
  d = dcol_ref[...] + 1.0
  di = lax.rsqrt(d)
  dinv_ref[...] = di
  dinv2_ref[...] = 1.0 / d
  u = h1_ref[...] * di
  ua_ref[...] = u[:, 0:16]
  ub_ref[...] = u[:, 16:32]
  uc_ref[...] = u[:, 32:48]
  ud_ref[...] = u[:, 48:64]


def _u1prep(dcol, h1):
  return pl.pallas_call(
      _u1prep_body,
      grid=(NRB,),
      in_specs=[
          pl.BlockSpec((RB, 1), lambda i: (i, 0)),
          pl.BlockSpec((RB, 64), lambda i: (i, 0)),
      ],
      out_specs=[
          pl.BlockSpec((RB, 1), lambda i: (i, 0)),
          pl.BlockSpec((RB, 1), lambda i: (i, 0)),
          pl.BlockSpec((RB, 16), lambda i: (i, 0)),
          pl.BlockSpec((RB, 16), lambda i: (i, 0)),
          pl.BlockSpec((RB, 16), lambda i: (i, 0)),
          pl.BlockSpec((RB, 16), lambda i: (i, 0)),
      ],
      out_shape=[
          jax.ShapeDtypeStruct((NP, 1), _f32),
          jax.ShapeDtypeStruct((NP, 1), _f32),
          jax.ShapeDtypeStruct((NP, 16), _f32),
          jax.ShapeDtypeStruct((NP, 16), _f32),
          jax.ShapeDtypeStruct((NP, 16), _f32),
          jax.ShapeDtypeStruct((NP, 16), _f32),
      ],
  )(dcol, h1)


def _epi1_body(a0_ref, a1_ref, a2_ref, a3_ref, h1_ref, dinv_ref, dinv2_ref,
               b_ref, w_ref, o1_ref, h2_ref, u2a_ref, u2b_ref):
  agg = jnp.concatenate(
      [a0_ref[...], a1_ref[...], a2_ref[...], a3_ref[...]], axis=1)
  o1 = jnp.maximum(
      dinv_ref[...] * agg + dinv2_ref[...] * h1_ref[...] + b_ref[...], 0.0)
  o1_ref[...] = o1
  h2 = jnp.dot(o1, w_ref[...], preferred_element_type=_f32)
  h2_ref[...] = h2
  u2 = h2 * dinv_ref[...]
  u2a_ref[...] = u2[:, 0:16]
  u2b_ref[...] = u2[:, 16:32]


def _epi1(aggs, h1, dinv, dinv2, b1r, w2t):
  return pl.pallas_call(
      _epi1_body,
      grid=(NRB,),
      in_specs=[
          pl.BlockSpec((RB, 16), lambda i: (i, 0)),
          pl.BlockSpec((RB, 16), lambda i: (i, 0)),
          pl.BlockSpec((RB, 16), lambda i: (i, 0)),
          pl.BlockSpec((RB, 16), lambda i: (i, 0)),
          pl.BlockSpec((RB, 64), lambda i: (i, 0)),
          pl.BlockSpec((RB, 1), lambda i: (i, 0)),
          pl.BlockSpec((RB, 1), lambda i: (i, 0)),
          pl.BlockSpec((1, 64), lambda i: (0, 0)),
          pl.BlockSpec((64, 32), lambda i: (0, 0)),
      ],
      out_specs=[
          pl.BlockSpec((RB, 64), lambda i: (i, 0)),
          pl.BlockSpec((RB, 32), lambda i: (i, 0)),
          pl.BlockSpec((RB, 16), lambda i: (i, 0)),
          pl.BlockSpec((RB, 16), lambda i: (i, 0)),
      ],
      out_shape=[
          jax.ShapeDtypeStruct((NP, 64), _f32),
          jax.ShapeDtypeStruct((NP, 32), _f32),
          jax.ShapeDtypeStruct((NP, 16), _f32),
          jax.ShapeDtypeStruct((NP, 16), _f32),
      ],
  )(*aggs, h1, dinv, dinv2, b1r, w2t)


def _epi2_body(a0_ref, a1_ref, h2_ref, dinv_ref, dinv2_ref, b_ref, w_ref,
               o2_ref, h3_ref, u3_ref):
  agg = jnp.concatenate([a0_ref[...], a1_ref[...]], axis=1)
  o2 = jnp.maximum(
      dinv_ref[...] * agg + dinv2_ref[...] * h2_ref[...] + b_ref[...], 0.0)
  o2_ref[...] = o2
  h3 = jnp.dot(o2, w_ref[...], preferred_element_type=_f32)
  h3_ref[...] = h3
  u3_ref[...] = h3 * dinv_ref[...]


def _epi2(aggs, h2, dinv, dinv2, b2r, w3t):
  return pl.pallas_call(
      _epi2_body,
      grid=(NRB,),
      in_specs=[
          pl.BlockSpec((RB, 16), lambda i: (i, 0)),
          pl.BlockSpec((RB, 16), lambda i: (i, 0)),
          pl.BlockSpec((RB, 32), lambda i: (i, 0)),
          pl.BlockSpec((RB, 1), lambda i: (i, 0)),
          pl.BlockSpec((RB, 1), lambda i: (i, 0)),
          pl.BlockSpec((1, 32), lambda i: (0, 0)),
          pl.BlockSpec((32, 16), lambda i: (0, 0)),
      ],
      out_specs=[
          pl.BlockSpec((RB, 32), lambda i: (i, 0)),
          pl.BlockSpec((RB, 16), lambda i: (i, 0)),
          pl.BlockSpec((RB, 16), lambda i: (i, 0)),
      ],
      out_shape=[
          jax.ShapeDtypeStruct((NP, 32), _f32),
          jax.ShapeDtypeStruct((NP, 16), _f32),
          jax.ShapeDtypeStruct((NP, 16), _f32),
      ],
  )(*aggs, h2, dinv, dinv2, b2r, w3t)


def _epi3_body(a_ref, h3_ref, dinv_ref, dinv2_ref, b_ref, o3_ref):
  agg = a_ref[...]
  o3_ref[...] = jnp.maximum(
      dinv_ref[...] * agg + dinv2_ref[...] * h3_ref[...] + b_ref[...], 0.0)


def _epi3(a, h3, dinv, dinv2, b3r):
  return pl.pallas_call(
      _epi3_body,
      grid=(NRB,),
      in_specs=[
          pl.BlockSpec((RB, 16), lambda i: (i, 0)),
          pl.BlockSpec((RB, 16), lambda i: (i, 0)),
          pl.BlockSpec((RB, 1), lambda i: (i, 0)),
          pl.BlockSpec((RB, 1), lambda i: (i, 0)),
          pl.BlockSpec((1, 16), lambda i: (0, 0)),
      ],
      out_specs=pl.BlockSpec((RB, 16), lambda i: (i, 0)),
      out_shape=jax.ShapeDtypeStruct((NP, 16), _f32),
  )(a, h3, dinv, dinv2, b3r)


def _head_body(px_ref, p1_ref, p2_ref, p3_ref, bng_ref, bnb_ref, bnm_ref,
               bnv_ref, f1w_ref, f1b_ref, f2w_ref, f2b_ref, out_ref):
  px = px_ref[...]
  sx = px[0] + px[1]
  s1 = p1_ref[...][0] + p1_ref[...][1]
  s2 = p2_ref[...][0] + p2_ref[...][1]
  s3 = p3_ref[...][0] + p3_ref[...][1]
  cnt = jnp.maximum(sx[:, 111:112], 1.0)
  agg = jnp.concatenate([sx[:, :111], s1, s2, s3], axis=1) / cnt
  a = ((agg - bnm_ref[...]) * lax.rsqrt(bnv_ref[...] + 1e-5) * bng_ref[...]
       + bnb_ref[...])
  z = jnp.maximum(
      jnp.dot(a, f1w_ref[...], preferred_element_type=_f32) + f1b_ref[...],
      0.0)
  z = jnp.dot(z, f2w_ref[...], preferred_element_type=_f32) + f2b_ref[...]
  out_ref[...] = jax.nn.sigmoid(z[:G, :])


def _head(px, p1, p2, p3, bng, bnb, bnm, bnv, f1w, f1b, f2w, f2b):
  return pl.pallas_call(
      _head_body,
      out_shape=jax.ShapeDtypeStruct((G, 1), _f32),
  )(px, p1, p2, p3, bng, bnb, bnm, bnv, f1w, f1b, f2w, f2b)


def kernel(x, edge_index, batches, W1, b1, W2, b2, W3, b3,
           bn_gamma, bn_beta, bn_mean, bn_var, fc1_W, fc1_b, fc2_W, fc2_b):
  xp = jnp.pad(
      jnp.concatenate([x, jnp.ones((N, 1), _f32)], axis=1),
      ((0, NP - N), (0, 0)))
  trash = PAD_NODE + jnp.arange(EP - E, dtype=jnp.int32) % (NP - N)
  src1 = jnp.concatenate([edge_index[0], trash])
  dst1 = jnp.concatenate([edge_index[1], trash])
  src2d = src1.reshape(EP // 128, 128)
  dst2d = dst1.reshape(EP // 128, 128)
  b2d = jnp.pad(batches, (0, NP - N), constant_values=G).reshape(NP // 128, 128)

  w1tp = jnp.pad(W1.T, ((0, 1), (0, 0)))
  w2t = W2.T
  w3t = W3.T
  b1r = b1.reshape(1, 64)
  b2r = b2.reshape(1, 32)
  b3r = b3.reshape(1, 16)
  f1w = fc1_W.T
  f1b = fc1_b.reshape(1, 64)
  f2w = fc2_W.T
  f2b = fc2_b.reshape(1, 1)
  bng = bn_gamma.reshape(1, 223)
  bnb = bn_beta.reshape(1, 223)
  bnm = bn_mean.reshape(1, 223)
  bnv = bn_var.reshape(1, 223)

  zeros_np = jnp.zeros((NP,), _f32)
  zgx = jnp.zeros((GSLICE, 112), _f32)
  zg1 = jnp.zeros((GSLICE, 64), _f32)
  zg2 = jnp.zeros((GSLICE, 32), _f32)
  zg3 = jnp.zeros((GSLICE, 16), _f32)

  h1 = _m1(xp, w1tp)
  degp = _deg(dst1, zeros_np)
  dsum = _dsum(degp.reshape(NT, NP))
  dinv, dinv2, u1a, u1b, u1c, u1d = _u1prep(dsum.reshape(NP, 1), h1)

  agg1 = _make_agg(16, [0, 1], [2, 3])([u1a, u1b, u1c, u1d], src2d, dst2d)
  o1, h2, u2a, u2b = _epi1(agg1, h1, dinv, dinv2, b1r, w2t)

  agg2 = _make_agg(16, [0], [1])([u2a, u2b], src2d, dst2d)
  o2, h3, u3 = _epi2(agg2, h2, dinv, dinv2, b2r, w3t)

  agg3 = _make_agg(16, [0], [])([u3], src2d, dst2d)
  o3 = _epi3(agg3[0], h3, dinv, dinv2, b3r)

  px, p1, p2, p3 = _pool(xp, o1, o2, o3, b2d, zgx, zg1, zg2, zg3)
  return _head(px, p1, p2, p3, bng, bnb, bnm, bnv, f1w, f1b, f2w, f2b)
